# baseline (speedup 1.0000x reference)
import functools

import jax
import jax.numpy as jnp
from jax import lax
from jax.experimental import pallas as pl
from jax.experimental.pallas import tpu as pltpu
from jax.experimental.pallas import tpu_sc as plsc

_TILES = 16
_CHUNK = 96
_LANES = 16


def _sc_agg(x2, edata, ew3, n_nodes, half):
    tiles, n_chunks_p, _, K = edata.shape
    n_chunks = n_chunks_p - 2
    assert n_chunks % 2 == 1, "pipeline peel assumes an odd chunk count"
    OC = 80
    assert n_nodes % OC == 0 and OC <= K
    out_chunks = n_nodes // OC
    out_rounds = -(-out_chunks // tiles)
    mesh = plsc.VectorSubcoreMesh(core_axis_name="c", subcore_axis_name="s")

    @functools.partial(
        pl.kernel,
        out_type=jax.ShapeDtypeStruct((2, n_nodes, half), jnp.float32),
        mesh=mesh,
        scratch_types=[
            pltpu.VMEM((2, K), jnp.int32),
            pltpu.VMEM((2, K), jnp.int32),
            pltpu.VMEM((1, K), jnp.float32),
            pltpu.VMEM((1, K), jnp.float32),
            pltpu.VMEM((K,), jnp.int32),
            pltpu.VMEM((K,), jnp.int32),
            pltpu.VMEM((K, half), jnp.float32),
            pltpu.VMEM((K, half), jnp.float32),
            pltpu.VMEM_SHARED((n_nodes, half), jnp.float32),
            pltpu.SemaphoreType.DMA,
            pltpu.SemaphoreType.DMA,
            pltpu.SemaphoreType.DMA,
            pltpu.SemaphoreType.DMA,
            pltpu.SemaphoreType.DMA,
            pltpu.SemaphoreType.DMA,
        ],
    )
    def k(x2_hbm, ed_hbm, ew_hbm, out_hbm,
          ebuf0, ebuf1, ewv0, ewv1, dst0, dst1, rows0, rows1,
          acc, sed0, sed1, sg0, sg1, ssc0, ssc1):
        sid = lax.axis_index("s")
        cid = lax.axis_index("c")
        EB, EW, DS, RW = ((ebuf0, ebuf1), (ewv0, ewv1),
                          (dst0, dst1), (rows0, rows1))
        SED, SG, SSC = (sed0, sed1), (sg0, sg1), (ssc0, ssc1)

        def issue_ed(c, p):
            pltpu.async_copy(ed_hbm.at[sid, c], EB[p], SED[p])
            pltpu.async_copy(ew_hbm.at[sid, pl.ds(c, 1)], EW[p], SED[p])

        def drain_ed(p):
            pltpu.make_async_copy(ed_hbm.at[sid, 0], EB[p], SED[p]).wait()
            pltpu.make_async_copy(ew_hbm.at[sid, pl.ds(0, 1)], EW[p],
                                  SED[p]).wait()

        def comp_idx(p):
            for j in range(K // _LANES):
                sl = pl.ds(j * _LANES, _LANES)
                v = EB[p][0, sl]
                EB[p][0, sl] = v + v + cid
                DS[p][sl] = EB[p][1, sl]

        def issue_gather(p):
            pltpu.async_copy(x2_hbm.at[EB[p].at[0]], RW[p], SG[p])

        def drain_gather(p):
            pltpu.make_async_copy(x2_hbm.at[EB[p].at[0]], RW[p],
                                  SG[p]).wait()

        def scale(p):
            def s_body(g, c2):
                w16 = EW[p][0, pl.ds(g * _LANES, _LANES)]
                row0 = g * _LANES
                for j in range(_LANES):
                    w = w16[j]
                    for q in range(half // _LANES):
                        sl = pl.ds(q * _LANES, _LANES)
                        RW[p][row0 + j, sl] = RW[p][row0 + j, sl] * w
                return c2
            lax.fori_loop(0, K // _LANES, s_body, None)

        def issue_scatter(p):
            pltpu.async_copy(RW[p], acc.at[DS[p]], SSC[p], add=True)

        def drain_scatter(p):
            pltpu.make_async_copy(RW[p], acc.at[DS[p]], SSC[p]).wait()

        def steady(c, b, drain_prev_scatter=True):
            o = 1 - b
            drain_ed(o)
            if drain_prev_scatter:
                drain_scatter(o)
            comp_idx(o)
            issue_gather(o)
            drain_gather(b)
            scale(b)
            issue_scatter(b)
            issue_ed(c + 2, b)

        def z_body(i, carry):
            for j in range(half // _LANES):
                sl = pl.ds(j * _LANES, _LANES)
                rows0[i, sl] = jnp.zeros((_LANES,), jnp.float32)
            return carry
        lax.fori_loop(0, K, z_body, None)

        def zc_body(i, carry):
            ch = sid + i * tiles

            @pl.when(ch < out_chunks)
            def _():
                off = pl.multiple_of(ch * OC, 8)
                pltpu.sync_copy(rows0.at[pl.ds(0, OC)], acc.at[pl.ds(off, OC)])
            return carry
        lax.fori_loop(0, out_rounds, zc_body, None)
        plsc.subcore_barrier()

        issue_ed(0, 0)
        issue_ed(1, 1)
        drain_ed(0)
        comp_idx(0)
        issue_gather(0)
        steady(0, 0, drain_prev_scatter=False)

        def pair_body(i, carry):
            c = 2 * i + 1
            steady(c, 1)
            steady(c + 1, 0)
            return carry
        lax.fori_loop(0, (n_chunks - 3) // 2, pair_body, None)

        steady(n_chunks - 2, 1)
        steady(n_chunks - 1, 0)

        drain_gather(1)
        drain_scatter(0)
        drain_ed(0)
        plsc.subcore_barrier()

        def o_body(i, carry):
            ch = sid + i * tiles

            @pl.when(ch < out_chunks)
            def _():
                off = pl.multiple_of(ch * OC, 8)
                pltpu.sync_copy(acc.at[pl.ds(off, OC)],
                                rows0.at[pl.ds(0, OC)])
                pltpu.sync_copy(rows0.at[pl.ds(0, OC)],
                                out_hbm.at[cid, pl.ds(off, OC)])
            return carry
        lax.fori_loop(0, out_rounds, o_body, None)

    return k(x2, edata, ew3)


def _tc_dense(agg2, x, wr0, wr1, wroot, brel, w2t, b2, block_rows=2000):
    n, feat = x.shape
    half = feat // 2
    hid = wroot.shape[1]
    ncls = w2t.shape[1]
    grid = n // block_rows

    def body(agg_ref, x_ref, wr0_ref, wr1_ref, wroot_ref, brel_ref,
             w2_ref, b2_ref, out_ref):
        h = jnp.dot(agg_ref[0], wr0_ref[...],
                    preferred_element_type=jnp.float32)
        h = h + jnp.dot(agg_ref[1], wr1_ref[...],
                        preferred_element_type=jnp.float32)
        h = h + jnp.dot(x_ref[...], wroot_ref[...],
                        preferred_element_type=jnp.float32)
        h = jnp.maximum(h + brel_ref[...], 0.0)
        logits = jnp.dot(h, w2_ref[...],
                         preferred_element_type=jnp.float32) + b2_ref[...]
        m = jnp.max(logits, axis=-1, keepdims=True)
        ls = logits - m
        out_ref[...] = ls - jnp.log(
            jnp.sum(jnp.exp(ls), axis=-1, keepdims=True))

    return pl.pallas_call(
        body,
        grid=(grid,),
        in_specs=[
            pl.BlockSpec((2, block_rows, half), lambda i: (0, i, 0)),
            pl.BlockSpec((block_rows, feat), lambda i: (i, 0)),
            pl.BlockSpec((half, hid), lambda i: (0, 0)),
            pl.BlockSpec((half, hid), lambda i: (0, 0)),
            pl.BlockSpec((feat, hid), lambda i: (0, 0)),
            pl.BlockSpec((1, hid), lambda i: (0, 0)),
            pl.BlockSpec((hid, ncls), lambda i: (0, 0)),
            pl.BlockSpec((1, ncls), lambda i: (0, 0)),
        ],
        out_specs=pl.BlockSpec((block_rows, ncls), lambda i: (i, 0)),
        out_shape=jax.ShapeDtypeStruct((n, ncls), jnp.float32),
    )(agg2, x, wr0, wr1, wroot, brel, w2t, b2)


def kernel(x, edge_weight, W_rel, b_rel, W_root, W2, b2, edge_index):
    n, feat = x.shape
    half = feat // 2
    e = edge_weight.shape[0]
    src = edge_index[0].astype(jnp.int32)
    dst = edge_index[1].astype(jnp.int32)

    per_tile = e // _TILES
    n_chunks = -(-per_tile // _CHUNK)
    if n_chunks % 2 == 0:
        n_chunks += 1
    pad = n_chunks * _CHUNK - per_tile
    x2 = x.reshape(2 * n, half)
    srcr = jnp.pad(src.reshape(_TILES, per_tile), ((0, 0), (0, pad)))
    dstr = jnp.pad(dst.reshape(_TILES, per_tile), ((0, 0), (0, pad)))
    edata = jnp.stack([srcr.reshape(_TILES, n_chunks, _CHUNK),
                       dstr.reshape(_TILES, n_chunks, _CHUNK)], axis=2)
    edata = jnp.pad(edata, ((0, 0), (0, 2), (0, 0), (0, 0)))
    ew3 = jnp.pad(edge_weight.reshape(_TILES, per_tile), ((0, 0), (0, pad)))
    ew3 = ew3.reshape(_TILES, n_chunks, _CHUNK)
    ew3 = jnp.pad(ew3, ((0, 0), (0, 2), (0, 0)))

    agg2 = _sc_agg(x2, edata, ew3, n, half)

    wrT = W_rel.T
    return _tc_dense(agg2, x, wrT[:half], wrT[half:], W_root.T,
                     b_rel.reshape(1, -1), W2.T, b2.reshape(1, -1))


# speedup vs baseline: 1.2326x; 1.2326x over previous
import functools

import jax
import jax.numpy as jnp
from jax import lax
from jax.experimental import pallas as pl
from jax.experimental.pallas import tpu as pltpu
from jax.experimental.pallas import tpu_sc as plsc

_TILES = 16
_CHUNK = 80
_LANES = 16


def _sc_agg(x2, edata, ew3, n_nodes, half):
    tiles, n_chunks_p, _, K = edata.shape
    n_chunks = n_chunks_p - 2
    assert n_chunks % 2 == 1, "pipeline peel assumes an odd chunk count"
    out_chunks = n_nodes // K
    out_rounds = -(-out_chunks // tiles)
    mesh = plsc.VectorSubcoreMesh(core_axis_name="c", subcore_axis_name="s")

    @functools.partial(
        pl.kernel,
        out_type=jax.ShapeDtypeStruct((2, n_nodes, half), jnp.float32),
        mesh=mesh,
        scratch_types=[
            pltpu.VMEM((2, K), jnp.int32),
            pltpu.VMEM((2, K), jnp.int32),
            pltpu.VMEM((1, K), jnp.float32),
            pltpu.VMEM((1, K), jnp.float32),
            pltpu.VMEM((K,), jnp.int32),
            pltpu.VMEM((K,), jnp.int32),
            pltpu.VMEM((K,), jnp.int32),
            pltpu.VMEM((K,), jnp.int32),
            pltpu.VMEM((K, half), jnp.float32),
            pltpu.VMEM((K, half), jnp.float32),
            pltpu.VMEM_SHARED((n_nodes, half), jnp.float32),
            pltpu.SemaphoreType.DMA,
            pltpu.SemaphoreType.DMA,
            pltpu.SemaphoreType.DMA,
            pltpu.SemaphoreType.DMA,
            pltpu.SemaphoreType.DMA,
            pltpu.SemaphoreType.DMA,
        ],
    )
    def k(x2_hbm, ed_hbm, ew_hbm, out_hbm,
          ebuf0, ebuf1, ewv0, ewv1, idx0, idx1, dst0, dst1, rows0, rows1,
          acc, sed0, sed1, sg0, sg1, ssc0, ssc1):
        sid = lax.axis_index("s")
        cid = lax.axis_index("c")
        EB, EW, IX, DS, RW = ((ebuf0, ebuf1), (ewv0, ewv1), (idx0, idx1),
                              (dst0, dst1), (rows0, rows1))
        SED, SG, SSC = (sed0, sed1), (sg0, sg1), (ssc0, ssc1)

        def issue_ed(c, p):
            pltpu.async_copy(ed_hbm.at[sid, c], EB[p], SED[p])
            pltpu.async_copy(ew_hbm.at[sid, pl.ds(c, 1)], EW[p], SED[p])

        def drain_ed(p):
            pltpu.make_async_copy(ed_hbm.at[sid, 0], EB[p], SED[p]).wait()
            pltpu.make_async_copy(ew_hbm.at[sid, pl.ds(0, 1)], EW[p],
                                  SED[p]).wait()

        def comp_idx(p):
            for j in range(K // _LANES):
                sl = pl.ds(j * _LANES, _LANES)
                v = EB[p][0, sl]
                IX[p][sl] = v + v + cid
                DS[p][sl] = EB[p][1, sl]

        def issue_gather(p):
            pltpu.async_copy(x2_hbm.at[IX[p]], RW[p], SG[p])

        def drain_gather(p):
            pltpu.make_async_copy(x2_hbm.at[IX[p]], RW[p], SG[p]).wait()

        def scale(p):
            def s_body(g, c2):
                w16 = EW[p][0, pl.ds(g * _LANES, _LANES)]
                row0 = g * _LANES
                for j in range(_LANES):
                    w = w16[j]
                    for q in range(half // _LANES):
                        sl = pl.ds(q * _LANES, _LANES)
                        RW[p][row0 + j, sl] = RW[p][row0 + j, sl] * w
                return c2
            lax.fori_loop(0, K // _LANES, s_body, None)

        def issue_scatter(p):
            pltpu.async_copy(RW[p], acc.at[DS[p]], SSC[p], add=True)

        def drain_scatter(p):
            pltpu.make_async_copy(RW[p], acc.at[DS[p]], SSC[p]).wait()

        def steady(c, b, drain_prev_scatter=True):
            o = 1 - b
            drain_ed(o)
            if drain_prev_scatter:
                drain_scatter(o)
            comp_idx(o)
            issue_gather(o)
            drain_gather(b)
            scale(b)
            issue_scatter(b)
            issue_ed(c + 2, b)

        def z_body(i, carry):
            for j in range(half // _LANES):
                sl = pl.ds(j * _LANES, _LANES)
                rows0[i, sl] = jnp.zeros((_LANES,), jnp.float32)
            return carry
        lax.fori_loop(0, K, z_body, None)

        def zc_body(i, carry):
            ch = sid + i * tiles

            @pl.when(ch < out_chunks)
            def _():
                off = pl.multiple_of(ch * K, 8)
                pltpu.sync_copy(rows0, acc.at[pl.ds(off, K)])
            return carry
        lax.fori_loop(0, out_rounds, zc_body, None)
        plsc.subcore_barrier()

        issue_ed(0, 0)
        issue_ed(1, 1)
        drain_ed(0)
        comp_idx(0)
        issue_gather(0)
        steady(0, 0, drain_prev_scatter=False)

        def pair_body(i, carry):
            c = 2 * i + 1
            steady(c, 1)
            steady(c + 1, 0)
            return carry
        lax.fori_loop(0, (n_chunks - 3) // 2, pair_body, None)

        steady(n_chunks - 2, 1)
        steady(n_chunks - 1, 0)

        drain_gather(1)
        drain_scatter(0)
        drain_ed(0)
        plsc.subcore_barrier()

        def o_body(i, carry):
            ch = sid + i * tiles

            @pl.when(ch < out_chunks)
            def _():
                off = pl.multiple_of(ch * K, 8)
                pltpu.sync_copy(acc.at[pl.ds(off, K)], rows0)
                pltpu.sync_copy(rows0, out_hbm.at[cid, pl.ds(off, K)])
            return carry
        lax.fori_loop(0, out_rounds, o_body, None)

    return k(x2, edata, ew3)


def _tc_dense(agg2, x, wr0, wr1, wroot, brel, w2t, b2, block_rows=2000):
    n, feat = x.shape
    half = feat // 2
    hid = wroot.shape[1]
    ncls = w2t.shape[1]
    grid = n // block_rows

    def body(agg_ref, x_ref, wr0_ref, wr1_ref, wroot_ref, brel_ref,
             w2_ref, b2_ref, out_ref):
        h = jnp.dot(agg_ref[0], wr0_ref[...],
                    preferred_element_type=jnp.float32)
        h = h + jnp.dot(agg_ref[1], wr1_ref[...],
                        preferred_element_type=jnp.float32)
        h = h + jnp.dot(x_ref[...], wroot_ref[...],
                        preferred_element_type=jnp.float32)
        h = jnp.maximum(h + brel_ref[...], 0.0)
        logits = jnp.dot(h, w2_ref[...],
                         preferred_element_type=jnp.float32) + b2_ref[...]
        m = jnp.max(logits, axis=-1, keepdims=True)
        ls = logits - m
        out_ref[...] = ls - jnp.log(
            jnp.sum(jnp.exp(ls), axis=-1, keepdims=True))

    return pl.pallas_call(
        body,
        grid=(grid,),
        in_specs=[
            pl.BlockSpec((2, block_rows, half), lambda i: (0, i, 0)),
            pl.BlockSpec((block_rows, feat), lambda i: (i, 0)),
            pl.BlockSpec((half, hid), lambda i: (0, 0)),
            pl.BlockSpec((half, hid), lambda i: (0, 0)),
            pl.BlockSpec((feat, hid), lambda i: (0, 0)),
            pl.BlockSpec((1, hid), lambda i: (0, 0)),
            pl.BlockSpec((hid, ncls), lambda i: (0, 0)),
            pl.BlockSpec((1, ncls), lambda i: (0, 0)),
        ],
        out_specs=pl.BlockSpec((block_rows, ncls), lambda i: (i, 0)),
        out_shape=jax.ShapeDtypeStruct((n, ncls), jnp.float32),
    )(agg2, x, wr0, wr1, wroot, brel, w2t, b2)


def kernel(x, edge_weight, W_rel, b_rel, W_root, W2, b2, edge_index):
    n, feat = x.shape
    half = feat // 2
    e = edge_weight.shape[0]
    src = edge_index[0].astype(jnp.int32)
    dst = edge_index[1].astype(jnp.int32)

    per_tile = e // _TILES
    n_chunks = per_tile // _CHUNK
    x2 = x.reshape(2 * n, half)
    edata = (jnp.stack([src, dst], axis=0)
             .reshape(2, _TILES, n_chunks, _CHUNK)
             .transpose(1, 2, 0, 3))
    edata = jnp.pad(edata, ((0, 0), (0, 2), (0, 0), (0, 0)))
    ew3 = edge_weight.reshape(_TILES, n_chunks, _CHUNK)
    ew3 = jnp.pad(ew3, ((0, 0), (0, 2), (0, 0)))

    agg2 = _sc_agg(x2, edata, ew3, n, half)

    wrT = W_rel.T
    return _tc_dense(agg2, x, wrT[:half], wrT[half:], W_root.T,
                     b_rel.reshape(1, -1), W2.T, b2.reshape(1, -1))
